# baseline (speedup 1.0000x reference)
import functools

import jax
import jax.numpy as jnp
from jax import lax
from jax.experimental import pallas as pl
from jax.experimental.pallas import tpu as pltpu
from jax.experimental.pallas import tpu_sc as plsc

BATCH = 16384
HIST = 50
DIM = 64

NC = 2
NS = 16
NW = NC * NS

CHUNK = 128
TOTAL = BATCH * HIST
N_CHUNKS = TOTAL // CHUNK
CPW = N_CHUNKS // NW
NBUF = 2
L = 16
NV = CHUNK // L


def _cbow_body(idx_hbm, table2_hbm, out_hbm, idx_v, pair_v, rows_v, t_v,
               gsems, ssems):
    wid = lax.axis_index("s") * NC + lax.axis_index("c")
    chunk0 = wid * CPW

    pltpu.sync_copy(idx_hbm.at[pl.ds(chunk0, CPW)], idx_v)

    def fire(t, b):
        for v in range(NV):
            idx16 = idx_v[t, pl.ds(v * L, L)]
            pair_v[b, pl.ds(v * L, L)] = lax.shift_right_logical(idx16, 1)
        pltpu.async_copy(table2_hbm.at[pair_v.at[b]],
                         rows_v.at[b, :, pl.ds(0, CHUNK)], gsems.at[b])

    def drain(b):
        pltpu.make_async_copy(table2_hbm.at[pl.ds(0, CHUNK)],
                              rows_v.at[b, :, pl.ds(0, CHUNK)],
                              gsems.at[b]).wait()

    def select_transpose(t, b):
        rows = rows_v.at[b]
        iota = lax.iota(jnp.int32, L)

        def d8_body(d8, _):
            for v in range(NV):
                row16 = iota + v * L
                col0 = lax.shift_left(
                    lax.bitwise_and(idx_v[t, pl.ds(v * L, L)], 1), 6)
                for dm in range(8):
                    val = plsc.load_gather(rows, [row16, col0 + d8 * 8 + dm])
                    t_v[b, d8, dm, pl.ds(v * L, L)] = val
            return _

        lax.fori_loop(0, 8, d8_body, None, unroll=False)

    def store(t, b):
        c = chunk0 + t
        h = lax.div(c, CHUNK)
        bc = lax.rem(c, CHUNK)
        pltpu.async_copy(t_v.at[b], out_hbm.at[h, :, bc], ssems.at[b])

    def store_wait(b):
        pltpu.make_async_copy(t_v.at[b], out_hbm.at[0, :, 0],
                              ssems.at[b]).wait()

    for b in range(NBUF):
        fire(b, b)

    def step(i, _):
        for b in range(NBUF):
            t = i * NBUF + b
            drain(b)

            @pl.when(t >= NBUF)
            def _():
                store_wait(b)

            select_transpose(t, b)
            store(t, b)

            @pl.when(t + NBUF < CPW)
            def _():
                fire(t + NBUF, b)
        return _

    lax.fori_loop(0, CPW // NBUF, step, None, unroll=False)

    for b in range(NBUF):
        store_wait(b)


@functools.partial(jax.jit, static_argnames=())
def kernel(input_ids, table):
    idx = input_ids.astype(jnp.int32).T.reshape(N_CHUNKS, CHUNK)
    table2 = table.reshape(500000, 128)
    mesh = plsc.VectorSubcoreMesh(core_axis_name="c", subcore_axis_name="s",
                                  num_cores=NC, num_subcores=NS)
    out5 = pl.kernel(
        _cbow_body,
        out_type=jax.ShapeDtypeStruct((HIST, 8, CHUNK, 8, CHUNK), jnp.float32),
        mesh=mesh,
        scratch_types=[
            pltpu.VMEM((CPW, CHUNK), jnp.int32),
            pltpu.VMEM((NBUF, CHUNK), jnp.int32),
            pltpu.VMEM((NBUF, CHUNK, CHUNK + 1), jnp.float32),
            pltpu.VMEM((NBUF, 8, 8, CHUNK), jnp.float32),
            pltpu.SemaphoreType.DMA((NBUF,)),
            pltpu.SemaphoreType.DMA((NBUF,)),
        ],
        compiler_params=pltpu.CompilerParams(needs_layout_passes=False),
    )(idx, table2)
    return out5.transpose(2, 4, 0, 1, 3).reshape(BATCH, HIST, DIM)


# speedup vs baseline: 1.4205x; 1.4205x over previous
import functools

import jax
import jax.numpy as jnp
from jax import lax
from jax.experimental import pallas as pl
from jax.experimental.pallas import tpu as pltpu
from jax.experimental.pallas import tpu_sc as plsc

BATCH = 16384
HIST = 50
DIM = 64

NC = 2
NS = 16
NW = NC * NS

CHUNK = 128
TOTAL = BATCH * HIST
N_CHUNKS = TOTAL // CHUNK
CPW = N_CHUNKS // NW
NBUF = 2
L = 16
NV = CHUNK // L


def _cbow_body(idx_hbm, table2_hbm, out_hbm, idx_v, pair_v, rows_v, t_v,
               gsems, ssems):
    wid = lax.axis_index("s") * NC + lax.axis_index("c")
    chunk0 = wid * CPW

    pltpu.sync_copy(idx_hbm.at[pl.ds(chunk0, CPW)], idx_v)

    def fire(t, b):
        for v in range(NV):
            idx16 = idx_v[t, pl.ds(v * L, L)]
            pair_v[b, pl.ds(v * L, L)] = lax.shift_right_logical(idx16, 1)
        pltpu.async_copy(table2_hbm.at[pair_v.at[b]],
                         rows_v.at[b, :, pl.ds(0, CHUNK)], gsems.at[b])

    def drain(b):
        pltpu.make_async_copy(table2_hbm.at[pl.ds(0, CHUNK)],
                              rows_v.at[b, :, pl.ds(0, CHUNK)],
                              gsems.at[b]).wait()

    def select_transpose(t, b):
        rows = rows_v.at[b]
        iota = lax.iota(jnp.int32, L)

        @plsc.parallel_loop(0, 8, unroll=2)
        def d8_body(d8):
            for v in range(NV):
                row16 = iota + v * L
                col0 = lax.shift_left(
                    lax.bitwise_and(idx_v[t, pl.ds(v * L, L)], 1), 6)
                for dm in range(8):
                    val = plsc.load_gather(rows, [row16, col0 + d8 * 8 + dm])
                    t_v[b, d8, dm, pl.ds(v * L, L)] = val

    def store(t, b):
        c = chunk0 + t
        h = lax.div(c, CHUNK)
        bc = lax.rem(c, CHUNK)
        pltpu.async_copy(t_v.at[b], out_hbm.at[h, :, bc], ssems.at[b])

    def store_wait(b):
        pltpu.make_async_copy(t_v.at[b], out_hbm.at[0, :, 0],
                              ssems.at[b]).wait()

    for b in range(NBUF):
        fire(b, b)

    def step(i, _):
        for b in range(NBUF):
            t = i * NBUF + b
            drain(b)

            @pl.when(t >= NBUF)
            def _():
                store_wait(b)

            select_transpose(t, b)
            store(t, b)

            @pl.when(t + NBUF < CPW)
            def _():
                fire(t + NBUF, b)
        return _

    lax.fori_loop(0, CPW // NBUF, step, None, unroll=False)

    for b in range(NBUF):
        store_wait(b)


@functools.partial(jax.jit, static_argnames=())
def kernel(input_ids, table):
    idx = input_ids.astype(jnp.int32).T.reshape(N_CHUNKS, CHUNK)
    table2 = table.reshape(500000, 128)
    mesh = plsc.VectorSubcoreMesh(core_axis_name="c", subcore_axis_name="s",
                                  num_cores=NC, num_subcores=NS)
    out5 = pl.kernel(
        _cbow_body,
        out_type=jax.ShapeDtypeStruct((HIST, 8, CHUNK, 8, CHUNK), jnp.float32),
        mesh=mesh,
        scratch_types=[
            pltpu.VMEM((CPW, CHUNK), jnp.int32),
            pltpu.VMEM((NBUF, CHUNK), jnp.int32),
            pltpu.VMEM((NBUF, CHUNK, CHUNK + 1), jnp.float32),
            pltpu.VMEM((NBUF, 8, 8, CHUNK), jnp.float32),
            pltpu.SemaphoreType.DMA((NBUF,)),
            pltpu.SemaphoreType.DMA((NBUF,)),
        ],
        compiler_params=pltpu.CompilerParams(needs_layout_passes=False),
    )(idx, table2)
    return out5.transpose(2, 4, 0, 1, 3).reshape(BATCH, HIST, DIM)


# speedup vs baseline: 1.4791x; 1.0412x over previous
import functools

import jax
import jax.numpy as jnp
from jax import lax
from jax.experimental import pallas as pl
from jax.experimental.pallas import tpu as pltpu
from jax.experimental.pallas import tpu_sc as plsc

BATCH = 16384
HIST = 50
DIM = 64

NC = 2
NS = 16
NW = NC * NS

CHUNK = 128
TOTAL = BATCH * HIST
N_CHUNKS = TOTAL // CHUNK
CPW = N_CHUNKS // NW
NBUF = 2
L = 16
NV = CHUNK // L


def _cbow_body(idx_hbm, table2_hbm, out_hbm, idx_v, pair_v, rows_v, t_v,
               gsems, ssems):
    wid = lax.axis_index("s") * NC + lax.axis_index("c")
    chunk0 = wid * CPW

    pltpu.sync_copy(idx_hbm.at[pl.ds(chunk0, CPW)], idx_v)

    def fire(t, b):
        for v in range(NV):
            idx16 = idx_v[t, pl.ds(v * L, L)]
            pair_v[b, pl.ds(v * L, L)] = lax.shift_right_logical(idx16, 1)
        pltpu.async_copy(table2_hbm.at[pair_v.at[b]],
                         rows_v.at[b, :, pl.ds(0, CHUNK)], gsems.at[b])

    def drain(b):
        pltpu.make_async_copy(table2_hbm.at[pl.ds(0, CHUNK)],
                              rows_v.at[b, :, pl.ds(0, CHUNK)],
                              gsems.at[b]).wait()

    def select_transpose(t, b):
        rows = rows_v.at[b]
        iota = lax.iota(jnp.int32, L)
        row16s = [iota + v * L for v in range(NV)]
        col0s = [lax.shift_left(
                     lax.bitwise_and(idx_v[t, pl.ds(v * L, L)], 1), 6)
                 for v in range(NV)]

        @plsc.parallel_loop(0, 8, unroll=4)
        def d8_body(d8):
            for v in range(NV):
                for dm in range(8):
                    val = plsc.load_gather(
                        rows, [row16s[v], col0s[v] + d8 * 8 + dm])
                    t_v[b, d8, dm, pl.ds(v * L, L)] = val

    def store(t, b):
        c = chunk0 + t
        h = lax.div(c, CHUNK)
        bc = lax.rem(c, CHUNK)
        pltpu.async_copy(t_v.at[b], out_hbm.at[h, :, bc], ssems.at[b])

    def store_wait(b):
        pltpu.make_async_copy(t_v.at[b], out_hbm.at[0, :, 0],
                              ssems.at[b]).wait()

    for b in range(NBUF):
        fire(b, b)

    def step(i, _):
        for b in range(NBUF):
            t = i * NBUF + b
            drain(b)

            @pl.when(t >= NBUF)
            def _():
                store_wait(b)

            select_transpose(t, b)
            store(t, b)

            @pl.when(t + NBUF < CPW)
            def _():
                fire(t + NBUF, b)
        return _

    lax.fori_loop(0, CPW // NBUF, step, None, unroll=False)

    for b in range(NBUF):
        store_wait(b)


@functools.partial(jax.jit, static_argnames=())
def kernel(input_ids, table):
    idx = input_ids.astype(jnp.int32).T.reshape(N_CHUNKS, CHUNK)
    table2 = table.reshape(500000, 128)
    mesh = plsc.VectorSubcoreMesh(core_axis_name="c", subcore_axis_name="s",
                                  num_cores=NC, num_subcores=NS)
    out5 = pl.kernel(
        _cbow_body,
        out_type=jax.ShapeDtypeStruct((HIST, 8, CHUNK, 8, CHUNK), jnp.float32),
        mesh=mesh,
        scratch_types=[
            pltpu.VMEM((CPW, CHUNK), jnp.int32),
            pltpu.VMEM((NBUF, CHUNK), jnp.int32),
            pltpu.VMEM((NBUF, CHUNK, CHUNK + 1), jnp.float32),
            pltpu.VMEM((NBUF, 8, 8, CHUNK), jnp.float32),
            pltpu.SemaphoreType.DMA((NBUF,)),
            pltpu.SemaphoreType.DMA((NBUF,)),
        ],
        compiler_params=pltpu.CompilerParams(needs_layout_passes=False),
    )(idx, table2)
    return out5.transpose(2, 4, 0, 1, 3).reshape(BATCH, HIST, DIM)


# speedup vs baseline: 1.4903x; 1.0075x over previous
import functools

import jax
import jax.numpy as jnp
from jax import lax
from jax.experimental import pallas as pl
from jax.experimental.pallas import tpu as pltpu
from jax.experimental.pallas import tpu_sc as plsc

BATCH = 16384
HIST = 50
DIM = 64

NC = 2
NS = 16
NW = NC * NS

CHUNK = 128
TOTAL = BATCH * HIST
N_CHUNKS = TOTAL // CHUNK
CPW = N_CHUNKS // NW
NBUF = 2
L = 16
NV = CHUNK // L


def _cbow_body(idx_hbm, table_hbm, out_hbm, idx_v, rows_v, t_v, gsems, ssems):
    wid = lax.axis_index("s") * NC + lax.axis_index("c")
    chunk0 = wid * CPW

    pltpu.sync_copy(idx_hbm.at[pl.ds(chunk0, CPW)], idx_v)

    def fire(t, b):
        pltpu.async_copy(table_hbm.at[idx_v.at[t]], rows_v.at[b], gsems.at[b])

    def drain(b):
        pltpu.make_async_copy(table_hbm.at[pl.ds(0, CHUNK)], rows_v.at[b],
                              gsems.at[b]).wait()

    def transpose(t, b):
        rows = rows_v.at[b]
        iota = lax.iota(jnp.int32, L)
        row16s = [iota + v * L for v in range(NV)]

        @plsc.parallel_loop(0, 8, unroll=4)
        def d8_body(d8):
            for v in range(NV):
                for dm in range(8):
                    val = plsc.load_gather(
                        rows, [row16s[v], jnp.full((L,), dm, jnp.int32)
                               + d8 * 8])
                    t_v[b, d8, dm, pl.ds(v * L, L)] = val

    def store(t, b):
        c = chunk0 + t
        h = lax.div(c, CHUNK)
        bc = lax.rem(c, CHUNK)
        pltpu.async_copy(t_v.at[b], out_hbm.at[h, :, bc], ssems.at[b])

    def store_wait(b):
        pltpu.make_async_copy(t_v.at[b], out_hbm.at[0, :, 0],
                              ssems.at[b]).wait()

    for b in range(NBUF):
        fire(b, b)

    def step(i, _):
        for b in range(NBUF):
            t = i * NBUF + b
            drain(b)

            @pl.when(t >= NBUF)
            def _():
                store_wait(b)

            transpose(t, b)
            store(t, b)

            @pl.when(t + NBUF < CPW)
            def _():
                fire(t + NBUF, b)
        return _

    lax.fori_loop(0, CPW // NBUF, step, None, unroll=False)

    for b in range(NBUF):
        store_wait(b)


@functools.partial(jax.jit, static_argnames=())
def kernel(input_ids, table):
    idx = input_ids.astype(jnp.int32).T.reshape(N_CHUNKS, CHUNK)
    mesh = plsc.VectorSubcoreMesh(core_axis_name="c", subcore_axis_name="s",
                                  num_cores=NC, num_subcores=NS)
    out5 = pl.kernel(
        _cbow_body,
        out_type=jax.ShapeDtypeStruct((HIST, 8, CHUNK, 8, CHUNK), jnp.float32),
        mesh=mesh,
        scratch_types=[
            pltpu.VMEM((CPW, CHUNK), jnp.int32),
            pltpu.VMEM((NBUF, CHUNK, DIM), jnp.float32),
            pltpu.VMEM((NBUF, 8, 8, CHUNK), jnp.float32),
            pltpu.SemaphoreType.DMA((NBUF,)),
            pltpu.SemaphoreType.DMA((NBUF,)),
        ],
        compiler_params=pltpu.CompilerParams(use_tc_tiling_on_sc=False,
                                             needs_layout_passes=False),
    )(idx, table)
    return out5.transpose(2, 4, 0, 1, 3).reshape(BATCH, HIST, DIM)


# speedup vs baseline: 1.5126x; 1.0150x over previous
import functools

import jax
import jax.numpy as jnp
from jax import lax
from jax.experimental import pallas as pl
from jax.experimental.pallas import tpu as pltpu
from jax.experimental.pallas import tpu_sc as plsc

BATCH = 16384
HIST = 50
DIM = 64

NC = 2
NS = 16
NW = NC * NS

CHUNK = 128
TOTAL = BATCH * HIST
N_CHUNKS = TOTAL // CHUNK
CPW = N_CHUNKS // NW
NBUF = 2
L = 16
NV = CHUNK // L


def _cbow_body(idx_hbm, table_hbm, out_hbm, idx_v, rows_v, t_v, gsems, ssems):
    wid = lax.axis_index("s") * NC + lax.axis_index("c")
    chunk0 = wid * CPW

    pltpu.sync_copy(idx_hbm.at[pl.ds(chunk0, CPW)], idx_v)

    def fire(t, b):
        pltpu.async_copy(table_hbm.at[idx_v.at[t]], rows_v.at[b], gsems.at[b])

    def drain(b):
        pltpu.make_async_copy(table_hbm.at[pl.ds(0, CHUNK)], rows_v.at[b],
                              gsems.at[b]).wait()

    def transpose(t, b):
        rows = rows_v.at[b]
        iota = lax.iota(jnp.int32, L)
        row16s = [iota + v * L for v in range(NV)]

        @plsc.parallel_loop(0, DIM, unroll=8)
        def d_body(d):
            d8 = lax.shift_right_logical(d, 3)
            dmoff = lax.shift_left(lax.bitwise_and(d, 7), 7)
            dcol = lax.broadcast(d, (L,))
            for v in range(NV):
                val = plsc.load_gather(rows, [row16s[v], dcol])
                t_v[b, d8, pl.ds(dmoff + v * L, L)] = val

    def store(t, b):
        c = chunk0 + t
        h = lax.div(c, CHUNK)
        bc = lax.rem(c, CHUNK)
        pltpu.async_copy(t_v.at[b], out_hbm.at[h, :, bc], ssems.at[b])

    def store_wait(b):
        pltpu.make_async_copy(t_v.at[b], out_hbm.at[0, :, 0],
                              ssems.at[b]).wait()

    for b in range(NBUF):
        fire(b, b)

    def step(i, _):
        for b in range(NBUF):
            t = i * NBUF + b
            drain(b)

            @pl.when(t >= NBUF)
            def _():
                store_wait(b)

            transpose(t, b)
            store(t, b)

            @pl.when(t + NBUF < CPW)
            def _():
                fire(t + NBUF, b)
        return _

    lax.fori_loop(0, CPW // NBUF, step, None, unroll=False)

    for b in range(NBUF):
        store_wait(b)


@functools.partial(jax.jit, static_argnames=())
def kernel(input_ids, table):
    idx = input_ids.astype(jnp.int32).T.reshape(N_CHUNKS, CHUNK)
    mesh = plsc.VectorSubcoreMesh(core_axis_name="c", subcore_axis_name="s",
                                  num_cores=NC, num_subcores=NS)
    out5 = pl.kernel(
        _cbow_body,
        out_type=jax.ShapeDtypeStruct((HIST, 8, CHUNK, 8 * CHUNK),
                                      jnp.float32),
        mesh=mesh,
        scratch_types=[
            pltpu.VMEM((CPW, CHUNK), jnp.int32),
            pltpu.VMEM((NBUF, CHUNK, DIM), jnp.float32),
            pltpu.VMEM((NBUF, 8, 8 * CHUNK), jnp.float32),
            pltpu.SemaphoreType.DMA((NBUF,)),
            pltpu.SemaphoreType.DMA((NBUF,)),
        ],
        compiler_params=pltpu.CompilerParams(use_tc_tiling_on_sc=False,
                                             needs_layout_passes=False),
    )(idx, table)
    return (out5.reshape(HIST, 8, CHUNK, 8, CHUNK)
            .transpose(2, 4, 0, 1, 3).reshape(BATCH, HIST, DIM))


# speedup vs baseline: 2.0851x; 1.3785x over previous
import functools

import jax
import jax.numpy as jnp
from jax import lax
from jax.experimental import pallas as pl
from jax.experimental.pallas import tpu as pltpu
from jax.experimental.pallas import tpu_sc as plsc

BATCH = 16384
HIST = 50
HPAD = 56
DIM = 64
DPAD = 128

NC = 2
NS = 16
NW = NC * NS

CHUNK = 128
TOTAL = BATCH * HIST
N_CHUNKS = TOTAL // CHUNK
CPW = N_CHUNKS // NW
NBUF = 2


def _cbow_body(idx_hbm, table_hbm, out_hbm, idx_v, rows_v, gsems, ssems):
    wid = lax.axis_index("s") * NC + lax.axis_index("c")
    chunk0 = wid * CPW

    pltpu.sync_copy(idx_hbm.at[pl.ds(chunk0, CPW)], idx_v)

    def fire(t, b):
        pltpu.async_copy(table_hbm.at[idx_v.at[t]], rows_v.at[b], gsems.at[b])

    def drain(b):
        pltpu.make_async_copy(table_hbm.at[pl.ds(0, CHUNK)], rows_v.at[b],
                              gsems.at[b]).wait()

    def store(t, b):
        c = chunk0 + t
        h = lax.div(c, CHUNK)
        bc = lax.rem(c, CHUNK)
        pltpu.async_copy(
            rows_v.at[b],
            out_hbm.at[pl.ds(bc * CHUNK, CHUNK), h, pl.ds(0, DIM)],
            ssems.at[b])

    def store_wait(b):
        pltpu.make_async_copy(rows_v.at[b],
                              out_hbm.at[pl.ds(0, CHUNK), 0, pl.ds(0, DIM)],
                              ssems.at[b]).wait()

    for b in range(NBUF):
        fire(b, b)

    def step(i, _):
        for b in range(NBUF):
            t = i * NBUF + b
            drain(b)

            @pl.when(t >= NBUF)
            def _():
                store_wait(b)

            store(t, b)

            @pl.when(t + NBUF < CPW)
            def _():
                fire(t + NBUF, b)
        return _

    lax.fori_loop(0, CPW // NBUF, step, None, unroll=False)

    for b in range(NBUF):
        store_wait(b)


@functools.partial(jax.jit, static_argnames=())
def kernel(input_ids, table):
    idx = input_ids.astype(jnp.int32).T.reshape(N_CHUNKS, CHUNK)
    mesh = plsc.VectorSubcoreMesh(core_axis_name="c", subcore_axis_name="s",
                                  num_cores=NC, num_subcores=NS)
    outp = pl.kernel(
        _cbow_body,
        out_type=jax.ShapeDtypeStruct((BATCH, HPAD, DPAD), jnp.float32),
        mesh=mesh,
        scratch_types=[
            pltpu.VMEM((CPW, CHUNK), jnp.int32),
            pltpu.VMEM((NBUF, CHUNK, DIM), jnp.float32),
            pltpu.SemaphoreType.DMA((NBUF,)),
            pltpu.SemaphoreType.DMA((NBUF,)),
        ],
        compiler_params=pltpu.CompilerParams(use_tc_tiling_on_sc=False,
                                             needs_layout_passes=False),
    )(idx, table)
    return outp[:, :HIST, :DIM]


# speedup vs baseline: 2.0951x; 1.0048x over previous
import functools

import jax
import jax.numpy as jnp
from jax import lax
from jax.experimental import pallas as pl
from jax.experimental.pallas import tpu as pltpu
from jax.experimental.pallas import tpu_sc as plsc

BATCH = 16384
HIST = 50
HPAD = 56
DIM = 64
DPAD = 128

NC = 2
NS = 16
NW = NC * NS

CHUNK = 128
TOTAL = BATCH * HIST
N_CHUNKS = TOTAL // CHUNK
CPW = N_CHUNKS // NW
NBUF = 4


def _cbow_body(idx_hbm, table_hbm, out_hbm, idx_v, rows_v, gsems, ssems):
    wid = lax.axis_index("s") * NC + lax.axis_index("c")
    chunk0 = wid * CPW

    pltpu.sync_copy(idx_hbm.at[pl.ds(chunk0, CPW)], idx_v)

    def fire(t, b):
        pltpu.async_copy(table_hbm.at[idx_v.at[t]], rows_v.at[b], gsems.at[b])

    def drain(b):
        pltpu.make_async_copy(table_hbm.at[pl.ds(0, CHUNK)], rows_v.at[b],
                              gsems.at[b]).wait()

    def store(t, b):
        c = chunk0 + t
        h = lax.div(c, CHUNK)
        bc = lax.rem(c, CHUNK)
        pltpu.async_copy(
            rows_v.at[b],
            out_hbm.at[pl.ds(bc * CHUNK, CHUNK), h, pl.ds(0, DIM)],
            ssems.at[b])

    def store_wait(b):
        pltpu.make_async_copy(rows_v.at[b],
                              out_hbm.at[pl.ds(0, CHUNK), 0, pl.ds(0, DIM)],
                              ssems.at[b]).wait()

    for b in range(NBUF):
        fire(b, b)

    def step(i, _):
        for b in range(NBUF):
            t = i * NBUF + b
            drain(b)

            @pl.when(t >= NBUF)
            def _():
                store_wait(b)

            store(t, b)

            @pl.when(t + NBUF < CPW)
            def _():
                fire(t + NBUF, b)
        return _

    lax.fori_loop(0, CPW // NBUF, step, None, unroll=False)

    for b in range(NBUF):
        store_wait(b)


@functools.partial(jax.jit, static_argnames=())
def kernel(input_ids, table):
    idx = input_ids.astype(jnp.int32).T.reshape(N_CHUNKS, CHUNK)
    mesh = plsc.VectorSubcoreMesh(core_axis_name="c", subcore_axis_name="s",
                                  num_cores=NC, num_subcores=NS)
    outp = pl.kernel(
        _cbow_body,
        out_type=jax.ShapeDtypeStruct((BATCH, HPAD, DPAD), jnp.float32),
        mesh=mesh,
        scratch_types=[
            pltpu.VMEM((CPW, CHUNK), jnp.int32),
            pltpu.VMEM((NBUF, CHUNK, DIM), jnp.float32),
            pltpu.SemaphoreType.DMA((NBUF,)),
            pltpu.SemaphoreType.DMA((NBUF,)),
        ],
        compiler_params=pltpu.CompilerParams(use_tc_tiling_on_sc=False,
                                             needs_layout_passes=False),
    )(idx, table)
    return outp[:, :HIST, :DIM]
